# baseline (speedup 1.0000x reference)
import jax
import jax.numpy as jnp
from jax import lax
from jax.experimental import pallas as pl
from jax.experimental.pallas import tpu as pltpu

N_ROWS = 32
ROWS = 16384
COLS = 128
CH = 1024
PER_ROW = ROWS // CH
NCH = N_ROWS * PER_ROW
NBUF = 32
W = 16
K = NBUF - W


def _read(x_hbm, src_hbm, buf, sem, i):
    r = i // PER_ROW
    sl = pl.ds((i % PER_ROW) * CH, CH)

    @pl.when(r == 0)
    def _():
        pltpu.make_async_copy(src_hbm.at[sl], buf, sem).start()

    @pl.when(r != 0)
    def _():
        pltpu.make_async_copy(x_hbm.at[r, sl], buf, sem).start()


def _wr(out_hbm, buf, sem, i):
    r = i // PER_ROW
    sl = pl.ds((i % PER_ROW) * CH, CH)
    return pltpu.make_async_copy(buf, out_hbm.at[r, sl], sem)


def _tc_body(x_hbm, src_hbm, out_hbm, *scratch):
    bufs = scratch[:NBUF]
    rsems = scratch[NBUF:2 * NBUF]
    wsems = scratch[2 * NBUF:]

    for j in range(K):
        _read(x_hbm, src_hbm, bufs[j], rsems[j], j)

    def body(g, carry):
        for b in range(NBUF):
            i = g * NBUF + b
            _wr(out_hbm, bufs[b], rsems[b], i).wait()
            _wr(out_hbm, bufs[b], wsems[b], i).start()
            bw = (b - W) % NBUF

            @pl.when(i >= W)
            def _():
                _wr(out_hbm, bufs[bw], wsems[bw], i - W).wait()

            br = (b + K) % NBUF

            @pl.when(i + K < NCH)
            def _():
                _read(x_hbm, src_hbm, bufs[br], rsems[br], i + K)
        return carry

    lax.fori_loop(0, NCH // NBUF, body, 0)
    for t in range(W):
        i = NCH - W + t
        _wr(out_hbm, bufs[i % NBUF], wsems[i % NBUF], i).wait()


def kernel(x, src):
    return pl.pallas_call(
        _tc_body,
        out_shape=jax.ShapeDtypeStruct(x.shape, x.dtype),
        in_specs=[pl.BlockSpec(memory_space=pltpu.MemorySpace.HBM)] * 2,
        out_specs=pl.BlockSpec(memory_space=pltpu.MemorySpace.HBM),
        scratch_shapes=(
            [pltpu.VMEM((CH, COLS), jnp.float32) for _ in range(NBUF)]
            + [pltpu.SemaphoreType.DMA for _ in range(2 * NBUF)]
        ),
    )(x, src)
